# baseline (speedup 1.0000x reference)
import functools

import jax
import jax.numpy as jnp
from jax import lax
from jax.experimental import pallas as pl
from jax.experimental.pallas import tpu as pltpu
from jax.experimental.pallas import tpu_sc as plsc

BATCH = 16384
EMB = 32
HID = 64

_NC = 2
_NS = 16
_NW = _NC * _NS
_BPW = BATCH // _NW
_CHUNK = 128
_NCHUNK = _BPW // _CHUNK


def _gather_body(sidx_h, ridx_h, stab_h, rtab_h, sout_h, rout_h,
                 sidx_v, ridx_v, srows_v, rrows_v, sem):
    wid = lax.axis_index("s") * _NC + lax.axis_index("c")
    base = wid * _BPW
    pltpu.sync_copy(sidx_h.at[pl.ds(wid * _NCHUNK, _NCHUNK)], sidx_v)
    pltpu.sync_copy(ridx_h.at[pl.ds(wid * _NCHUNK, _NCHUNK)], ridx_v)
    copies = []
    for j in range(_NCHUNK):
        copies.append(pltpu.async_copy(
            stab_h.at[sidx_v.at[j]],
            srows_v.at[pl.ds(j * _CHUNK, _CHUNK)], sem))
        copies.append(pltpu.async_copy(
            rtab_h.at[ridx_v.at[j]],
            rrows_v.at[pl.ds(j * _CHUNK, _CHUNK)], sem))
    for c in copies:
        c.wait()
    pltpu.sync_copy(srows_v, sout_h.at[pl.ds(base, _BPW)])
    pltpu.sync_copy(rrows_v, rout_h.at[pl.ds(base, _BPW)])


_gather_call = functools.partial(
    pl.kernel,
    out_type=[jax.ShapeDtypeStruct((BATCH, EMB), jnp.float32),
              jax.ShapeDtypeStruct((BATCH, EMB), jnp.float32)],
    mesh=plsc.VectorSubcoreMesh(core_axis_name="c", subcore_axis_name="s"),
    scratch_types=[pltpu.VMEM((_NCHUNK, _CHUNK), jnp.int32),
                   pltpu.VMEM((_NCHUNK, _CHUNK), jnp.int32),
                   pltpu.VMEM((_BPW, EMB), jnp.float32),
                   pltpu.VMEM((_BPW, EMB), jnp.float32),
                   pltpu.SemaphoreType.DMA],
    compiler_params=pltpu.CompilerParams(use_tc_tiling_on_sc=False),
)(_gather_body)


_BLK = 1024


def _mlp_body(s_ref, r_ref, w1s_ref, w1r_ref, b1_ref, w2_ref, b2_ref, out_ref):
    h = jnp.tanh(
        jnp.dot(s_ref[...], w1s_ref[...], preferred_element_type=jnp.float32)
        + jnp.dot(r_ref[...], w1r_ref[...], preferred_element_type=jnp.float32)
        + b1_ref[...])
    logit = jnp.sum(h * w2_ref[...], axis=1) + b2_ref[0, 0]
    out_ref[...] = jax.nn.sigmoid(logit)


def _mlp_call(s_emb, r_emb, w1s, w1r, b1, w2, b2):
    grid = BATCH // _BLK
    return pl.pallas_call(
        _mlp_body,
        grid=(grid,),
        in_specs=[
            pl.BlockSpec((_BLK, EMB), lambda i: (i, 0)),
            pl.BlockSpec((_BLK, EMB), lambda i: (i, 0)),
            pl.BlockSpec((EMB, HID), lambda i: (0, 0)),
            pl.BlockSpec((EMB, HID), lambda i: (0, 0)),
            pl.BlockSpec((1, HID), lambda i: (0, 0)),
            pl.BlockSpec((1, HID), lambda i: (0, 0)),
            pl.BlockSpec((1, 1), lambda i: (0, 0)),
        ],
        out_specs=pl.BlockSpec((_BLK,), lambda i: (i,)),
        out_shape=jax.ShapeDtypeStruct((BATCH,), jnp.float32),
    )(s_emb, r_emb, w1s, w1r, b1, w2, b2)


def kernel(sender_idx_batch, receiver_idx_batch, sender_table, receiver_table,
           W1, b1, W2, b2):
    sidx = sender_idx_batch.astype(jnp.int32).reshape(_NW * _NCHUNK, _CHUNK)
    ridx = receiver_idx_batch.astype(jnp.int32).reshape(_NW * _NCHUNK, _CHUNK)
    s_emb, r_emb = _gather_call(sidx, ridx, sender_table, receiver_table)
    w1s = W1[:, :EMB].T
    w1r = W1[:, EMB:].T
    b1r = b1.reshape(1, HID)
    w2r = W2.reshape(1, HID)
    b2r = b2.reshape(1, 1)
    return _mlp_call(s_emb, r_emb, w1s, w1r, b1r, w2r, b2r)
